# baseline (speedup 1.0000x reference)
import functools

import jax
import jax.numpy as jnp
import numpy as np
from jax import lax
from jax.experimental import pallas as pl
from jax.experimental.pallas import tpu as pltpu
from jax.experimental.pallas import tpu_sc as plsc

EMB = 1000000
DIM = 32
BATCH = 16384
NF = 26

NC = 2
NS = 16
NW = NC * NS
BPW = BATCH // NW
CHUNK = 128
NCHUNK = BPW // CHUNK
IDX_PER_CHUNK = CHUNK * NF

TROW = NF * DIM // 8
QDIM = BATCH // CHUNK * 8

_U = np.uint32
_M_HI = _U(17179)
_M_LO = _U(56963)


def _mod1e6(h):
    a = h >> _U(16)
    b = h & _U(0xFFFF)
    mid1 = a * _M_LO
    mid2 = b * _M_HI
    lo = b * _M_LO
    t = (lo >> _U(16)) + (mid1 & _U(0xFFFF)) + (mid2 & _U(0xFFFF))
    hi = a * _M_HI + (mid1 >> _U(16)) + (mid2 >> _U(16)) + (t >> _U(16))
    q = hi >> _U(18)
    return h - q * _U(1000000)


@functools.partial(
    pl.kernel,
    out_type=jax.ShapeDtypeStruct((TROW, QDIM, CHUNK), jnp.float32),
    mesh=plsc.VectorSubcoreMesh(core_axis_name="c", subcore_axis_name="s"),
    compiler_params=pltpu.CompilerParams(
        needs_layout_passes=False, use_tc_tiling_on_sc=False
    ),
    scratch_types=[
        pltpu.VMEM((BPW,), jnp.int32),
        pltpu.VMEM((NF,), jnp.int32),
        pltpu.VMEM((NF, CHUNK), jnp.int32),
        pltpu.VMEM((IDX_PER_CHUNK, DIM), jnp.float32),
        pltpu.VMEM((2, 4, 8, CHUNK), jnp.float32),
        pltpu.SemaphoreType.DMA,
        pltpu.SemaphoreType.DMA,
    ],
)
def _emb_lookup(
    x_hbm, fnum_hbm, table_hbm, out_hbm, x_v, f_v, idx_v, g_v, b_v, gsem, osem
):
    wid = lax.axis_index("s") * NC + lax.axis_index("c")
    base = wid * BPW
    pltpu.sync_copy(x_hbm.at[pl.ds(base, BPW)], x_v)
    pltpu.sync_copy(fnum_hbm, f_v)
    lane = lax.iota(jnp.int32, 16)
    m2c = [jnp.full((16,), m2, jnp.int32) for m2 in range(DIM)]

    def chunk_body(c, carry):
        xk = [
            x_v[pl.ds(c * CHUNK + k * 16, 16)].astype(_U) * _U(2654435761)
            for k in range(CHUNK // 16)
        ]

        def field_hash(j, carry2):
            fj = plsc.load_gather(f_v, [jnp.full((16,), j, jnp.int32)])
            cj = fj.astype(_U) * _U(40503) + _U(2166136261)
            for k in range(CHUNK // 16):
                h = xk[k] + cj
                h = (h ^ (h >> _U(15))) * _U(2246822519)
                h = h ^ (h >> _U(13))
                idx_v[j, pl.ds(k * 16, 16)] = _mod1e6(h).astype(jnp.int32)
            return carry2

        lax.fori_loop(0, NF, field_hash, 0)

        copies = [
            pltpu.async_copy(
                table_hbm.at[idx_v.at[g]],
                g_v.at[pl.ds(g * CHUNK, CHUNK)],
                gsem,
            )
            for g in range(NF)
        ]
        for cp in copies:
            cp.wait()

        q0 = (wid * NCHUNK + c) * 8

        def out_slice(j):
            return out_hbm.at[pl.ds(4 * j, 4), pl.ds(q0, 8), :]

        def field_out(j, carry2):
            par = j % 2

            @pl.when(j >= 2)
            def _drain():
                pltpu.make_async_copy(b_v.at[par], out_slice(j - 2), osem).wait()

            jrow = j * CHUNK
            rowv = [jrow + (b0 * 16 + lane) for b0 in range(CHUNK // 16)]
            for m2 in range(DIM):
                for b0 in range(CHUNK // 16):
                    v = plsc.load_gather(g_v, [rowv[b0], m2c[m2]])
                    b_v[par, m2 // 8, m2 % 8, pl.ds(b0 * 16, 16)] = v
            pltpu.async_copy(b_v.at[par], out_slice(j), osem)
            return carry2

        lax.fori_loop(0, NF, field_out, 0)
        pltpu.make_async_copy(b_v.at[0], out_slice(NF - 2), osem).wait()
        pltpu.make_async_copy(b_v.at[1], out_slice(NF - 1), osem).wait()
        return carry

    lax.fori_loop(0, NCHUNK, chunk_body, 0)


def kernel(x, fnum, table):
    out = _emb_lookup(x, fnum, table)
    return (
        out.reshape(TROW, BATCH // CHUNK, 8, CHUNK)
        .transpose(1, 3, 0, 2)
        .reshape(BATCH, NF * DIM)
    )
